# Initial kernel scaffold; baseline (speedup 1.0000x reference)
import functools

import jax
import jax.numpy as jnp
from jax import lax
from jax.experimental import pallas as pl
from jax.experimental.pallas import tpu as pltpu
from jax.experimental.pallas import tpu_sc as plsc

E_DIM = 128
G_DIM = 128
N_GRAPHS = 64
N_NODES = 10000
N_EDGES = 320000
LANES = 16
NFV = E_DIM // LANES

NC = 2
NS = 16
NW = NC * NS
EPW = N_EDGES // NW
CHUNK = 400
NCHUNK = EPW // CHUNK


def _sc_body(eidx_hbm, e_hbm, batch_hbm, stats_out, cnt_out,
             batch_v, idx_v, eb_v, ech_v, sum_v, min_v, max_v, sq_v, cnt_v):
    c = lax.axis_index("c")
    s = lax.axis_index("s")
    wid = s * NC + c
    base = wid * EPW

    zeros = jnp.zeros((LANES,), jnp.float32)
    ones = jnp.ones((LANES,), jnp.float32)
    pinf = jnp.full((LANES,), jnp.inf, jnp.float32)
    ninf = jnp.full((LANES,), -jnp.inf, jnp.float32)

    def init_row(g, carry):
        for f in range(NFV):
            sl = pl.ds(f * LANES, LANES)
            sum_v[g, sl] = zeros
            min_v[g, sl] = pinf
            max_v[g, sl] = ninf
            sq_v[g, sl] = zeros
        cnt_v[g, :] = zeros
        return carry

    lax.fori_loop(0, N_GRAPHS, init_row, None)

    pltpu.sync_copy(batch_hbm, batch_v)

    def chunk_body(ci, carry):
        cb = base + ci * CHUNK
        pltpu.sync_copy(eidx_hbm.at[1, pl.ds(cb, CHUNK)], idx_v)
        pltpu.sync_copy(e_hbm.at[pl.ds(cb, CHUNK)], ech_v)

        def gather_body(j, carry2):
            idx = idx_v[pl.ds(j * LANES, LANES)]
            ids = plsc.load_gather(batch_v, [idx])
            eb_v[pl.ds(j * LANES, LANES)] = ids
            return carry2

        lax.fori_loop(0, CHUNK // LANES, gather_body, None)

        def edge_body(i, carry2):
            b = eb_v[i]
            plsc.addupdate(cnt_v.at[b, :], ones)
            for f in range(NFV):
                sl = pl.ds(f * LANES, LANES)
                v = ech_v[i, sl]
                plsc.addupdate(sum_v.at[b, sl], v)
                plsc.addupdate(sq_v.at[b, sl], v * v)
                min_v[b, sl] = jnp.minimum(min_v[b, sl], v)
                max_v[b, sl] = jnp.maximum(max_v[b, sl], v)
            return carry2

        lax.fori_loop(0, CHUNK, edge_body, None)
        return carry

    lax.fori_loop(0, NCHUNK, chunk_body, None)

    pltpu.sync_copy(sum_v, stats_out.at[wid, 0])
    pltpu.sync_copy(min_v, stats_out.at[wid, 1])
    pltpu.sync_copy(max_v, stats_out.at[wid, 2])
    pltpu.sync_copy(sq_v, stats_out.at[wid, 3])
    pltpu.sync_copy(cnt_v, cnt_out.at[wid])


_sc_reduce = functools.partial(
    pl.kernel,
    out_type=(
        jax.ShapeDtypeStruct((NW, 4, N_GRAPHS, E_DIM), jnp.float32),
        jax.ShapeDtypeStruct((NW, N_GRAPHS, LANES), jnp.float32),
    ),
    mesh=plsc.VectorSubcoreMesh(core_axis_name="c", subcore_axis_name="s"),
    scratch_types=[
        pltpu.VMEM((N_NODES,), jnp.int32),
        pltpu.VMEM((CHUNK,), jnp.int32),
        pltpu.VMEM((CHUNK,), jnp.int32),
        pltpu.VMEM((CHUNK, E_DIM), jnp.float32),
        pltpu.VMEM((N_GRAPHS, E_DIM), jnp.float32),
        pltpu.VMEM((N_GRAPHS, E_DIM), jnp.float32),
        pltpu.VMEM((N_GRAPHS, E_DIM), jnp.float32),
        pltpu.VMEM((N_GRAPHS, E_DIM), jnp.float32),
        pltpu.VMEM((N_GRAPHS, LANES), jnp.float32),
    ],
)(_sc_body)


def _tc_body(stats_ref, cnt_ref, w_ref, b_ref, out_ref):
    stats = stats_ref[...]
    counts = jnp.sum(cnt_ref[...][:, :, 0], axis=0)[:, None]
    sums = jnp.sum(stats[:, 0], axis=0)
    mins = jnp.min(stats[:, 1], axis=0)
    maxs = jnp.max(stats[:, 2], axis=0)
    sq = jnp.sum(stats[:, 3], axis=0)

    e1 = sums / jnp.maximum(counts, 1.0)
    nonempty = counts > 0
    e2 = jnp.where(nonempty, mins, 0.0)
    e3 = jnp.where(nonempty, maxs, 0.0)
    var = jnp.maximum(sq - counts * e1 * e1, 0.0) / jnp.maximum(counts - 1.0, 1.0)
    e4 = jnp.sqrt(var)
    g = jnp.concatenate([e1, e2, e3, e4], axis=1)
    out = lax.dot_general(g, w_ref[...], (((1,), (1,)), ((), ())),
                          preferred_element_type=jnp.float32)
    out_ref[...] = out + b_ref[...][None, :]


def _tc_finalize(stats, cnt, W, b):
    return pl.pallas_call(
        _tc_body,
        out_shape=jax.ShapeDtypeStruct((N_GRAPHS, G_DIM), jnp.float32),
    )(stats, cnt, W, b)


def kernel(e_index, e, batch, W, b):
    stats, cnt = _sc_reduce(e_index, e, batch)
    return _tc_finalize(stats, cnt, W, b)


# speedup vs baseline: 9.4382x; 9.4382x over previous
import functools

import jax
import jax.numpy as jnp
from jax import lax
from jax.experimental import pallas as pl
from jax.experimental.pallas import tpu as pltpu
from jax.experimental.pallas import tpu_sc as plsc

E_DIM = 128
G_DIM = 128
N_GRAPHS = 64
N_NODES = 10000
N_EDGES = 320000
LANES = 16
NFV = E_DIM // LANES

NC = 2
NS = 16
NW = NC * NS
EPW = N_EDGES // NW
CHUNK = 400
NCHUNK = EPW // CHUNK


def _sc_body(eidx_hbm, e_hbm, batch_hbm, stats_out, cnt_out,
             batch_v, idx_v, ech_v, sum_v, min_v, max_v, sq_v, cnt_v):
    c = lax.axis_index("c")
    s = lax.axis_index("s")
    wid = s * NC + c
    base = wid * EPW

    zeros = jnp.zeros((LANES,), jnp.float32)
    ones = jnp.ones((LANES,), jnp.float32)
    pinf = jnp.full((LANES,), jnp.inf, jnp.float32)
    ninf = jnp.full((LANES,), -jnp.inf, jnp.float32)

    def init_row(g, carry):
        for f in range(NFV):
            sl = pl.ds(f * LANES, LANES)
            sum_v[g, sl] = zeros
            min_v[g, sl] = pinf
            max_v[g, sl] = ninf
            sq_v[g, sl] = zeros
        cnt_v[g, :] = zeros
        return carry

    lax.fori_loop(0, N_GRAPHS, init_row, None)

    pltpu.sync_copy(batch_hbm, batch_v)

    def chunk_body(ci, carry):
        cb = base + ci * CHUNK
        pltpu.sync_copy(eidx_hbm.at[pl.ds(cb, CHUNK)], idx_v)
        pltpu.sync_copy(e_hbm.at[pl.ds(cb, CHUNK)], ech_v)

        def blk_body(j, carry2):
            idx = idx_v[pl.ds(j * LANES, LANES)]
            ids = plsc.load_gather(batch_v, [idx])
            for k in range(LANES):
                b = ids[k]
                i = j * LANES + k
                plsc.addupdate(cnt_v.at[b, :], ones)
                for f in range(NFV):
                    sl = pl.ds(f * LANES, LANES)
                    v = ech_v[i, sl]
                    plsc.addupdate(sum_v.at[b, sl], v)
                    plsc.addupdate(sq_v.at[b, sl], v * v)
                    min_v[b, sl] = jnp.minimum(min_v[b, sl], v)
                    max_v[b, sl] = jnp.maximum(max_v[b, sl], v)
            return carry2

        lax.fori_loop(0, CHUNK // LANES, blk_body, None)
        return carry

    lax.fori_loop(0, NCHUNK, chunk_body, None)

    pltpu.sync_copy(sum_v, stats_out.at[wid, 0])
    pltpu.sync_copy(min_v, stats_out.at[wid, 1])
    pltpu.sync_copy(max_v, stats_out.at[wid, 2])
    pltpu.sync_copy(sq_v, stats_out.at[wid, 3])
    pltpu.sync_copy(cnt_v, cnt_out.at[wid])


_sc_reduce = functools.partial(
    pl.kernel,
    out_type=(
        jax.ShapeDtypeStruct((NW, 4, N_GRAPHS, E_DIM), jnp.float32),
        jax.ShapeDtypeStruct((NW, N_GRAPHS, LANES), jnp.float32),
    ),
    mesh=plsc.VectorSubcoreMesh(core_axis_name="c", subcore_axis_name="s"),
    compiler_params=pltpu.CompilerParams(needs_layout_passes=False),
    scratch_types=[
        pltpu.VMEM((N_NODES,), jnp.int32),
        pltpu.VMEM((CHUNK,), jnp.int32),
        pltpu.VMEM((CHUNK, E_DIM), jnp.float32),
        pltpu.VMEM((N_GRAPHS, E_DIM), jnp.float32),
        pltpu.VMEM((N_GRAPHS, E_DIM), jnp.float32),
        pltpu.VMEM((N_GRAPHS, E_DIM), jnp.float32),
        pltpu.VMEM((N_GRAPHS, E_DIM), jnp.float32),
        pltpu.VMEM((N_GRAPHS, LANES), jnp.float32),
    ],
)(_sc_body)


def _tc_body(stats_ref, cnt_ref, w_ref, b_ref, out_ref):
    stats = stats_ref[...]
    counts = jnp.sum(cnt_ref[...][:, :, 0], axis=0)[:, None]
    sums = jnp.sum(stats[:, 0], axis=0)
    mins = jnp.min(stats[:, 1], axis=0)
    maxs = jnp.max(stats[:, 2], axis=0)
    sq = jnp.sum(stats[:, 3], axis=0)

    e1 = sums / jnp.maximum(counts, 1.0)
    nonempty = counts > 0
    e2 = jnp.where(nonempty, mins, 0.0)
    e3 = jnp.where(nonempty, maxs, 0.0)
    var = jnp.maximum(sq - counts * e1 * e1, 0.0) / jnp.maximum(counts - 1.0, 1.0)
    e4 = jnp.sqrt(var)
    g = jnp.concatenate([e1, e2, e3, e4], axis=1)
    out = lax.dot_general(g, w_ref[...], (((1,), (1,)), ((), ())),
                          preferred_element_type=jnp.float32)
    out_ref[...] = out + b_ref[...][None, :]


def _tc_finalize(stats, cnt, W, b):
    return pl.pallas_call(
        _tc_body,
        out_shape=jax.ShapeDtypeStruct((N_GRAPHS, G_DIM), jnp.float32),
    )(stats, cnt, W, b)


def kernel(e_index, e, batch, W, b):
    stats, cnt = _sc_reduce(e_index[1], e, batch)
    return _tc_finalize(stats, cnt, W, b)


# speedup vs baseline: 13.2340x; 1.4022x over previous
import functools

import jax
import jax.numpy as jnp
from jax import lax
from jax.experimental import pallas as pl
from jax.experimental.pallas import tpu as pltpu
from jax.experimental.pallas import tpu_sc as plsc

E_DIM = 128
G_DIM = 128
N_GRAPHS = 64
N_NODES = 10000
N_EDGES = 320000
LANES = 16
NFV = E_DIM // LANES

NC = 2
NS = 16
NW = NC * NS
EPW = N_EDGES // NW
CHUNK = 400
NCHUNK = EPW // CHUNK


def _sc_body(eidx_hbm, e_hbm, batch_hbm, stats_out, cnt_out,
             batch_v, idx_v, ech_v, sum_v, min_v, max_v, sq_v, cnt_v):
    c = lax.axis_index("c")
    s = lax.axis_index("s")
    wid = s * NC + c
    base = wid * EPW

    zeros = jnp.zeros((LANES,), jnp.float32)
    ones = jnp.ones((LANES,), jnp.float32)
    pinf = jnp.full((LANES,), jnp.inf, jnp.float32)
    ninf = jnp.full((LANES,), -jnp.inf, jnp.float32)

    def init_row(g, carry):
        for f in range(NFV):
            sl = pl.ds(f * LANES, LANES)
            sum_v[g, sl] = zeros
            min_v[g, sl] = pinf
            max_v[g, sl] = ninf
            sq_v[g, sl] = zeros
        cnt_v[g, :] = zeros
        return carry

    lax.fori_loop(0, N_GRAPHS, init_row, None)

    pltpu.sync_copy(batch_hbm, batch_v)

    def chunk_body(ci, carry):
        cb = base + ci * CHUNK
        pltpu.sync_copy(eidx_hbm.at[pl.ds(cb, CHUNK)], idx_v)
        pltpu.sync_copy(e_hbm.at[pl.ds(cb, CHUNK)], ech_v)

        def blk_body(j, carry2):
            idx = idx_v[pl.ds(j * LANES, LANES)]
            ids = plsc.load_gather(batch_v, [idx])
            for k in range(LANES):
                b = ids[k]
                i = j * LANES + k
                sls = [pl.ds(f * LANES, LANES) for f in range(NFV)]
                vs = [ech_v[i, sl] for sl in sls]
                mns = [min_v[b, sl] for sl in sls]
                mxs = [max_v[b, sl] for sl in sls]
                plsc.addupdate(cnt_v.at[b, :], ones)
                for f in range(NFV):
                    sl = sls[f]
                    v = vs[f]
                    plsc.addupdate(sum_v.at[b, sl], v)
                    plsc.addupdate(sq_v.at[b, sl], v * v)
                    min_v[b, sl] = jnp.minimum(mns[f], v)
                    max_v[b, sl] = jnp.maximum(mxs[f], v)
            return carry2

        lax.fori_loop(0, CHUNK // LANES, blk_body, None)
        return carry

    lax.fori_loop(0, NCHUNK, chunk_body, None)

    pltpu.sync_copy(sum_v, stats_out.at[wid, 0])
    pltpu.sync_copy(min_v, stats_out.at[wid, 1])
    pltpu.sync_copy(max_v, stats_out.at[wid, 2])
    pltpu.sync_copy(sq_v, stats_out.at[wid, 3])
    pltpu.sync_copy(cnt_v, cnt_out.at[wid])


_sc_reduce = functools.partial(
    pl.kernel,
    out_type=(
        jax.ShapeDtypeStruct((NW, 4, N_GRAPHS, E_DIM), jnp.float32),
        jax.ShapeDtypeStruct((NW, N_GRAPHS, LANES), jnp.float32),
    ),
    mesh=plsc.VectorSubcoreMesh(core_axis_name="c", subcore_axis_name="s"),
    compiler_params=pltpu.CompilerParams(needs_layout_passes=False),
    scratch_types=[
        pltpu.VMEM((N_NODES,), jnp.int32),
        pltpu.VMEM((CHUNK,), jnp.int32),
        pltpu.VMEM((CHUNK, E_DIM), jnp.float32),
        pltpu.VMEM((N_GRAPHS, E_DIM), jnp.float32),
        pltpu.VMEM((N_GRAPHS, E_DIM), jnp.float32),
        pltpu.VMEM((N_GRAPHS, E_DIM), jnp.float32),
        pltpu.VMEM((N_GRAPHS, E_DIM), jnp.float32),
        pltpu.VMEM((N_GRAPHS, LANES), jnp.float32),
    ],
)(_sc_body)


def _tc_body(stats_ref, cnt_ref, w_ref, b_ref, out_ref):
    stats = stats_ref[...]
    counts = jnp.sum(cnt_ref[...][:, :, 0], axis=0)[:, None]
    sums = jnp.sum(stats[:, 0], axis=0)
    mins = jnp.min(stats[:, 1], axis=0)
    maxs = jnp.max(stats[:, 2], axis=0)
    sq = jnp.sum(stats[:, 3], axis=0)

    e1 = sums / jnp.maximum(counts, 1.0)
    nonempty = counts > 0
    e2 = jnp.where(nonempty, mins, 0.0)
    e3 = jnp.where(nonempty, maxs, 0.0)
    var = jnp.maximum(sq - counts * e1 * e1, 0.0) / jnp.maximum(counts - 1.0, 1.0)
    e4 = jnp.sqrt(var)
    g = jnp.concatenate([e1, e2, e3, e4], axis=1)
    out = lax.dot_general(g, w_ref[...], (((1,), (1,)), ((), ())),
                          preferred_element_type=jnp.float32)
    out_ref[...] = out + b_ref[...][None, :]


def _tc_finalize(stats, cnt, W, b):
    return pl.pallas_call(
        _tc_body,
        out_shape=jax.ShapeDtypeStruct((N_GRAPHS, G_DIM), jnp.float32),
    )(stats, cnt, W, b)


def kernel(e_index, e, batch, W, b):
    stats, cnt = _sc_reduce(e_index[1], e, batch)
    return _tc_finalize(stats, cnt, W, b)


# speedup vs baseline: 16.3878x; 1.2383x over previous
import functools

import jax
import jax.numpy as jnp
from jax import lax
from jax.experimental import pallas as pl
from jax.experimental.pallas import tpu as pltpu
from jax.experimental.pallas import tpu_sc as plsc

E_DIM = 128
G_DIM = 128
N_GRAPHS = 64
N_NODES = 10000
N_EDGES = 320000
LANES = 16
NFV = E_DIM // LANES

NC = 2
NS = 16
NW = NC * NS
EPW = N_EDGES // NW
CHUNK = 400
NCHUNK = EPW // CHUNK

EB_BLK = 2000
NB = N_EDGES // EB_BLK


def _eb_body(eidx_hbm, batch_hbm, out_hbm, batch_v, idx_v, eb_v):
    c = lax.axis_index("c")
    s = lax.axis_index("s")
    wid = s * NC + c
    base = wid * EPW
    pltpu.sync_copy(batch_hbm, batch_v)
    pltpu.sync_copy(eidx_hbm.at[pl.ds(base, EPW)], idx_v)

    def gather_body(j, carry):
        idx = idx_v[pl.ds(j * LANES, LANES)]
        ids = plsc.load_gather(batch_v, [idx])
        eb_v[pl.ds(j * LANES, LANES)] = ids
        return carry

    lax.fori_loop(0, EPW // LANES, gather_body, None)
    pltpu.sync_copy(eb_v, out_hbm.at[pl.ds(base, EPW)])


_eb_gather = functools.partial(
    pl.kernel,
    out_type=jax.ShapeDtypeStruct((N_EDGES,), jnp.int32),
    mesh=plsc.VectorSubcoreMesh(core_axis_name="c", subcore_axis_name="s"),
    compiler_params=pltpu.CompilerParams(needs_layout_passes=False),
    scratch_types=[
        pltpu.VMEM((N_NODES,), jnp.int32),
        pltpu.VMEM((EPW,), jnp.int32),
        pltpu.VMEM((EPW,), jnp.int32),
    ],
)(_eb_body)


def _lin_body(eb_ref, e_ref, sq_ref, cnt_ref):
    @pl.when(pl.program_id(0) == 0)
    def _():
        sq_ref[...] = jnp.zeros_like(sq_ref)
        cnt_ref[...] = jnp.zeros_like(cnt_ref)

    eb = eb_ref[...].reshape(1, EB_BLK)
    gids = lax.broadcasted_iota(jnp.int32, (N_GRAPHS, 1), 0)
    oht = (eb == gids).astype(jnp.float32)
    e = e_ref[...]
    esq = jnp.concatenate([e, e * e], axis=1)
    acc = lax.dot_general(oht, esq, (((1,), (0,)), ((), ())),
                          preferred_element_type=jnp.float32)
    sq_ref[...] += acc
    cnt_ref[...] += jnp.sum(oht, axis=1)[:, None]


def _lin_reduce(ebr, e):
    return pl.pallas_call(
        _lin_body,
        grid=(NB,),
        in_specs=[
            pl.BlockSpec((1, 1, EB_BLK), lambda i: (i, 0, 0)),
            pl.BlockSpec((EB_BLK, E_DIM), lambda i: (i, 0)),
        ],
        out_specs=[
            pl.BlockSpec((N_GRAPHS, 2 * E_DIM), lambda i: (0, 0)),
            pl.BlockSpec((N_GRAPHS, 1), lambda i: (0, 0)),
        ],
        out_shape=[
            jax.ShapeDtypeStruct((N_GRAPHS, 2 * E_DIM), jnp.float32),
            jax.ShapeDtypeStruct((N_GRAPHS, 1), jnp.float32),
        ],
    )(ebr, e)


def _mm_body(eidx_hbm, e_hbm, batch_hbm, stats_out,
             batch_v, idx_v, ech_v, min_v, max_v):
    c = lax.axis_index("c")
    s = lax.axis_index("s")
    wid = s * NC + c
    base = wid * EPW

    pinf = jnp.full((LANES,), jnp.inf, jnp.float32)
    ninf = jnp.full((LANES,), -jnp.inf, jnp.float32)

    def init_row(g, carry):
        for f in range(NFV):
            sl = pl.ds(f * LANES, LANES)
            min_v[g, sl] = pinf
            max_v[g, sl] = ninf
        return carry

    lax.fori_loop(0, N_GRAPHS, init_row, None)

    pltpu.sync_copy(batch_hbm, batch_v)

    def chunk_body(ci, carry):
        cb = base + ci * CHUNK
        pltpu.sync_copy(eidx_hbm.at[pl.ds(cb, CHUNK)], idx_v)
        pltpu.sync_copy(e_hbm.at[pl.ds(cb, CHUNK)], ech_v)

        def blk_body(j, carry2):
            idx = idx_v[pl.ds(j * LANES, LANES)]
            ids = plsc.load_gather(batch_v, [idx])
            for k in range(LANES):
                b = ids[k]
                i = j * LANES + k
                sls = [pl.ds(f * LANES, LANES) for f in range(NFV)]
                vs = [ech_v[i, sl] for sl in sls]
                mns = [min_v[b, sl] for sl in sls]
                mxs = [max_v[b, sl] for sl in sls]
                for f in range(NFV):
                    sl = sls[f]
                    v = vs[f]
                    min_v[b, sl] = jnp.minimum(mns[f], v)
                    max_v[b, sl] = jnp.maximum(mxs[f], v)
            return carry2

        lax.fori_loop(0, CHUNK // LANES, blk_body, None)
        return carry

    lax.fori_loop(0, NCHUNK, chunk_body, None)

    pltpu.sync_copy(min_v, stats_out.at[wid, 0])
    pltpu.sync_copy(max_v, stats_out.at[wid, 1])


_mm_reduce = functools.partial(
    pl.kernel,
    out_type=jax.ShapeDtypeStruct((NW, 2, N_GRAPHS, E_DIM), jnp.float32),
    mesh=plsc.VectorSubcoreMesh(core_axis_name="c", subcore_axis_name="s"),
    compiler_params=pltpu.CompilerParams(needs_layout_passes=False),
    scratch_types=[
        pltpu.VMEM((N_NODES,), jnp.int32),
        pltpu.VMEM((CHUNK,), jnp.int32),
        pltpu.VMEM((CHUNK, E_DIM), jnp.float32),
        pltpu.VMEM((N_GRAPHS, E_DIM), jnp.float32),
        pltpu.VMEM((N_GRAPHS, E_DIM), jnp.float32),
    ],
)(_mm_body)


def _fin_body(mm_ref, sq_ref, cnt_ref, w_ref, b_ref, out_ref):
    mm = mm_ref[...]
    counts = cnt_ref[...]
    sums = sq_ref[...][:, :E_DIM]
    sq = sq_ref[...][:, E_DIM:]
    mins = jnp.min(mm[:, 0], axis=0)
    maxs = jnp.max(mm[:, 1], axis=0)

    e1 = sums / jnp.maximum(counts, 1.0)
    nonempty = counts > 0
    e2 = jnp.where(nonempty, mins, 0.0)
    e3 = jnp.where(nonempty, maxs, 0.0)
    var = jnp.maximum(sq - counts * e1 * e1, 0.0) / jnp.maximum(counts - 1.0, 1.0)
    e4 = jnp.sqrt(var)
    g = jnp.concatenate([e1, e2, e3, e4], axis=1)
    out = lax.dot_general(g, w_ref[...], (((1,), (1,)), ((), ())),
                          preferred_element_type=jnp.float32)
    out_ref[...] = out + b_ref[...][None, :]


def _tc_finalize(mm, sq, cnt, W, b):
    return pl.pallas_call(
        _fin_body,
        out_shape=jax.ShapeDtypeStruct((N_GRAPHS, G_DIM), jnp.float32),
    )(mm, sq, cnt, W, b)


def kernel(e_index, e, batch, W, b):
    ei1 = e_index[1]
    eb = _eb_gather(ei1, batch)
    ebr = eb.reshape(NB, 1, EB_BLK)
    sq, cnt = _lin_reduce(ebr, e)
    mm = _mm_reduce(ei1, e, batch)
    return _tc_finalize(mm, sq, cnt, W, b)


# speedup vs baseline: 16.3962x; 1.0005x over previous
import functools

import jax
import jax.numpy as jnp
from jax import lax
from jax.experimental import pallas as pl
from jax.experimental.pallas import tpu as pltpu
from jax.experimental.pallas import tpu_sc as plsc

E_DIM = 128
G_DIM = 128
N_GRAPHS = 64
N_NODES = 10000
N_EDGES = 320000
LANES = 16
NFV = E_DIM // LANES

NC = 2
NS = 16
NW = NC * NS
EPW = N_EDGES // NW
CHUNK = 400
NCHUNK = EPW // CHUNK

EB_BLK = 2000
NB = N_EDGES // EB_BLK


def _eb_body(eidx_hbm, batch_hbm, out_hbm, batch_v, idx_v, eb_v):
    c = lax.axis_index("c")
    s = lax.axis_index("s")
    wid = s * NC + c
    base = wid * EPW
    pltpu.sync_copy(batch_hbm, batch_v)
    pltpu.sync_copy(eidx_hbm.at[pl.ds(base, EPW)], idx_v)

    def gather_body(j, carry):
        idx = idx_v[pl.ds(j * LANES, LANES)]
        ids = plsc.load_gather(batch_v, [idx])
        eb_v[pl.ds(j * LANES, LANES)] = ids
        return carry

    lax.fori_loop(0, EPW // LANES, gather_body, None)
    pltpu.sync_copy(eb_v, out_hbm.at[pl.ds(base, EPW)])


_eb_gather = functools.partial(
    pl.kernel,
    out_type=jax.ShapeDtypeStruct((N_EDGES,), jnp.int32),
    mesh=plsc.VectorSubcoreMesh(core_axis_name="c", subcore_axis_name="s"),
    compiler_params=pltpu.CompilerParams(needs_layout_passes=False),
    scratch_types=[
        pltpu.VMEM((N_NODES,), jnp.int32),
        pltpu.VMEM((EPW,), jnp.int32),
        pltpu.VMEM((EPW,), jnp.int32),
    ],
)(_eb_body)


def _lin_body(eb_ref, e_ref, sq_ref, cnt_ref):
    @pl.when(pl.program_id(0) == 0)
    def _():
        sq_ref[...] = jnp.zeros_like(sq_ref)
        cnt_ref[...] = jnp.zeros_like(cnt_ref)

    eb = eb_ref[...].reshape(1, EB_BLK)
    gids = lax.broadcasted_iota(jnp.int32, (N_GRAPHS, 1), 0)
    oht = (eb == gids).astype(jnp.float32)
    e = e_ref[...]
    esq = jnp.concatenate([e, e * e], axis=1)
    acc = lax.dot_general(oht, esq, (((1,), (0,)), ((), ())),
                          preferred_element_type=jnp.float32)
    sq_ref[...] += acc
    cnt_ref[...] += jnp.sum(oht, axis=1)[:, None]


def _lin_reduce(ebr, e):
    return pl.pallas_call(
        _lin_body,
        grid=(NB,),
        in_specs=[
            pl.BlockSpec((1, 1, EB_BLK), lambda i: (i, 0, 0)),
            pl.BlockSpec((EB_BLK, E_DIM), lambda i: (i, 0)),
        ],
        out_specs=[
            pl.BlockSpec((N_GRAPHS, 2 * E_DIM), lambda i: (0, 0)),
            pl.BlockSpec((N_GRAPHS, 1), lambda i: (0, 0)),
        ],
        out_shape=[
            jax.ShapeDtypeStruct((N_GRAPHS, 2 * E_DIM), jnp.float32),
            jax.ShapeDtypeStruct((N_GRAPHS, 1), jnp.float32),
        ],
    )(ebr, e)


def _mm_body(eidx_hbm, e_hbm, batch_hbm, stats_out,
             batch_v, idx_v, ech_v, min_v, max_v):
    c = lax.axis_index("c")
    s = lax.axis_index("s")
    wid = s * NC + c
    base = wid * EPW

    pinf = jnp.full((LANES,), jnp.inf, jnp.float32)
    ninf = jnp.full((LANES,), -jnp.inf, jnp.float32)

    def init_row(g, carry):
        for f in range(NFV):
            sl = pl.ds(f * LANES, LANES)
            min_v[g, sl] = pinf
            max_v[g, sl] = ninf
        return carry

    lax.fori_loop(0, N_GRAPHS, init_row, None)

    pltpu.sync_copy(batch_hbm, batch_v)

    def chunk_body(ci, carry):
        cb = base + ci * CHUNK
        pltpu.sync_copy(eidx_hbm.at[pl.ds(cb, CHUNK)], idx_v)
        pltpu.sync_copy(e_hbm.at[pl.ds(cb, CHUNK)], ech_v)

        def blk_body(j, carry2):
            idx = idx_v[pl.ds(j * LANES, LANES)]
            ids = plsc.load_gather(batch_v, [idx])
            for k in range(LANES):
                b = ids[k]
                i = j * LANES + k
                sls = [pl.ds(f * LANES, LANES) for f in range(NFV)]
                vs = [ech_v[i, sl] for sl in sls]
                mns = [min_v[b, sl] for sl in sls]
                mxs = [max_v[b, sl] for sl in sls]
                for f in range(NFV):
                    sl = sls[f]
                    v = vs[f]
                    min_v[b, sl] = jnp.minimum(mns[f], v)
                    max_v[b, sl] = jnp.maximum(mxs[f], v)
            return carry2

        lax.fori_loop(0, CHUNK // LANES, blk_body, None)
        return carry

    lax.fori_loop(0, NCHUNK, chunk_body, None)

    pltpu.sync_copy(min_v, stats_out.at[wid, 0])
    pltpu.sync_copy(max_v, stats_out.at[wid, 1])


_mm_reduce = functools.partial(
    pl.kernel,
    out_type=jax.ShapeDtypeStruct((NW, 2, N_GRAPHS, E_DIM), jnp.float32),
    mesh=plsc.VectorSubcoreMesh(core_axis_name="c", subcore_axis_name="s"),
    compiler_params=pltpu.CompilerParams(needs_layout_passes=False),
    scratch_types=[
        pltpu.VMEM((N_NODES,), jnp.int32),
        pltpu.VMEM((CHUNK,), jnp.int32),
        pltpu.VMEM((CHUNK, E_DIM), jnp.float32),
        pltpu.VMEM((N_GRAPHS, E_DIM), jnp.float32),
        pltpu.VMEM((N_GRAPHS, E_DIM), jnp.float32),
    ],
)(_mm_body)


def _fin_body(mm_ref, sq_ref, cnt_ref, w_ref, b_ref, out_ref):
    mm = mm_ref[...]
    counts = cnt_ref[...]
    sums = sq_ref[...][:, :E_DIM]
    sq = sq_ref[...][:, E_DIM:]
    mins = jnp.min(mm[:, 0], axis=0)
    maxs = jnp.max(mm[:, 1], axis=0)

    e1 = sums / jnp.maximum(counts, 1.0)
    nonempty = counts > 0
    e2 = jnp.where(nonempty, mins, 0.0)
    e3 = jnp.where(nonempty, maxs, 0.0)
    var = jnp.maximum(sq - counts * e1 * e1, 0.0) / jnp.maximum(counts - 1.0, 1.0)
    e4 = jnp.sqrt(var)
    g = jnp.concatenate([e1, e2, e3, e4], axis=1)
    out = lax.dot_general(g, w_ref[...], (((1,), (1,)), ((), ())),
                          preferred_element_type=jnp.float32)
    out_ref[...] = out + b_ref[...][None, :]


def _tc_finalize(mm, sq, cnt, W, b):
    return pl.pallas_call(
        _fin_body,
        out_shape=jax.ShapeDtypeStruct((N_GRAPHS, G_DIM), jnp.float32),
    )(mm, sq, cnt, W, b)


def kernel(e_index, e, batch, W, b):
    ei1 = e_index[1]
    mm = _mm_reduce(ei1, e, batch)
    eb = _eb_gather(ei1, batch)
    ebr = eb.reshape(NB, 1, EB_BLK)
    sq, cnt = _lin_reduce(ebr, e)
    return _tc_finalize(mm, sq, cnt, W, b)
